# baseline (speedup 1.0000x reference)
import functools

import jax
import jax.numpy as jnp
from jax import lax
from jax.experimental import pallas as pl
from jax.experimental.pallas import tpu as pltpu
from jax.experimental.pallas import tpu_sc as plsc

_NC = 2
_NS = 16
_NW = _NC * _NS
_C = 128
_EPS = 1e-5


def _mesh():
    return plsc.VectorSubcoreMesh(core_axis_name="c", subcore_axis_name="s")


def _sc_degree(dst2, ones_blk, zrow):
    cpw = dst2.shape[0] // _NW
    rpt = zrow.shape[0]
    npad = rpt * _NS
    dh = zrow.shape[1]

    @functools.partial(
        pl.kernel,
        out_type=jax.ShapeDtypeStruct((_NC, npad, dh), jnp.float32),
        mesh=_mesh(),
        scratch_types=[
            pltpu.VMEM((cpw, _C), jnp.int32),
            pltpu.VMEM((_C, dh), jnp.float32),
            pltpu.VMEM_SHARED((npad, dh), jnp.float32),
        ],
    )
    def degk(dst_hbm, ones_hbm, z_hbm, out_hbm, dstv, onev, acc):
        cid = lax.axis_index("c")
        sid = lax.axis_index("s")
        wid = sid * _NC + cid
        pltpu.sync_copy(z_hbm, acc.at[pl.ds(sid * rpt, rpt)])
        pltpu.sync_copy(ones_hbm, onev)
        pltpu.sync_copy(dst_hbm.at[pl.ds(wid * cpw, cpw)], dstv)
        plsc.subcore_barrier()

        @pl.loop(0, cpw)
        def _(ci):
            pltpu.sync_copy(onev, acc.at[dstv.at[ci]], add=True)

        plsc.subcore_barrier()
        pltpu.sync_copy(acc.at[pl.ds(sid * rpt, rpt)],
                        out_hbm.at[cid, pl.ds(sid * rpt, rpt)])

    return degk(dst2, ones_blk, zrow)


def _sc_edge(hp, src2, dst2, zrow):
    dh = hp.shape[1]
    cpw = src2.shape[0] // _NW
    rpt = zrow.shape[0]
    npad = rpt * _NS
    cpf = _NC * cpw

    chf = 64

    @functools.partial(
        pl.kernel,
        out_type=jax.ShapeDtypeStruct((_NC, npad, dh), jnp.float32),
        mesh=_mesh(),
        scratch_types=[
            pltpu.VMEM((chf, _C), jnp.int32),
            pltpu.VMEM((_C,), jnp.int32),
            pltpu.VMEM((_C,), jnp.int32),
            pltpu.VMEM((_C, dh), jnp.float32),
            pltpu.VMEM((_C, dh), jnp.float32),
            pltpu.VMEM_SHARED((npad, dh), jnp.float32),
            pltpu.SemaphoreType.DMA,
            pltpu.SemaphoreType.DMA,
        ],
    )
    def ek(hp_hbm, src_hbm, dst_hbm, z_hbm, out_hbm,
           dstv, s0, s1, buf0, buf1, acc, g0, g1):
        cid = lax.axis_index("c")
        sid = lax.axis_index("s")
        base = sid * cpf
        pltpu.sync_copy(z_hbm, acc.at[pl.ds(sid * rpt, rpt)])
        plsc.subcore_barrier()

        @pl.when(cid == 0)
        def _():
            @pl.loop(0, cpf, step=2)
            def _(ci):
                li = lax.rem(ci, chf)

                @pl.when(li == 0)
                def _():
                    off = pl.multiple_of(base + ci, 8)
                    pltpu.sync_copy(dst_hbm.at[pl.ds(off, chf)], dstv)

                pltpu.sync_copy(src_hbm.at[base + ci], s0)
                da = pltpu.async_copy(hp_hbm.at[s0], buf0, g0)
                pltpu.sync_copy(src_hbm.at[base + ci + 1], s1)
                db = pltpu.async_copy(hp_hbm.at[s1], buf1, g1)
                da.wait()
                pltpu.sync_copy(buf0, acc.at[dstv.at[li]], add=True)
                db.wait()
                pltpu.sync_copy(buf1, acc.at[dstv.at[li + 1]], add=True)

        plsc.subcore_barrier()
        pltpu.sync_copy(acc.at[pl.ds(sid * rpt, rpt)],
                        out_hbm.at[cid, pl.ds(sid * rpt, rpt)])

    return ek(hp, src2, dst2, zrow)


def _tc_init(x, w0, degp):
    n, d = x.shape
    h = w0.shape[1]

    def body(x_ref, w_ref, dp_ref, dinv_ref, hp_ref):
        deg = 1.0 + dp_ref[0][:n, 0:1] + dp_ref[1][:n, 0:1]
        dinv = 1.0 / jnp.sqrt(deg)
        dinv_ref[...] = dinv
        hp_ref[...] = jnp.dot(x_ref[...], w_ref[...],
                              preferred_element_type=jnp.float32,
                              precision=lax.Precision.HIGHEST) * dinv

    return pl.pallas_call(
        body,
        out_shape=(jax.ShapeDtypeStruct((n, 1), jnp.float32),
                   jax.ShapeDtypeStruct((n, h), jnp.float32)),
    )(x, w0, degp)


def _bn_relu_res(p_ref, hp, dinv, xres, b, g, be):
    n = hp.shape[0]
    scat = p_ref[0][:n, :] + p_ref[1][:n, :]
    t = dinv * (scat + hp) + b
    m = jnp.mean(t, axis=0, keepdims=True)
    v = jnp.mean((t - m) ** 2, axis=0, keepdims=True)
    hh = g * (t - m) / jnp.sqrt(v + _EPS) + be
    return xres + jnp.maximum(hh, 0.0)


def _tc_bn(p, hp, dinv, xres, b, g, be):
    n, h = hp.shape

    def body(p_ref, hp_ref, dinv_ref, x_ref, b_ref, g_ref, be_ref, xn_ref):
        xn_ref[...] = _bn_relu_res(p_ref, hp_ref[...], dinv_ref[...],
                                   x_ref[...], b_ref[...], g_ref[...],
                                   be_ref[...])

    return pl.pallas_call(
        body,
        out_shape=jax.ShapeDtypeStruct((n, h), jnp.float32),
    )(p, hp, dinv, xres, b, g, be)


def _tc_mm(xin, w, dinv):
    n, h = xin.shape[0], w.shape[1]

    def body(x_ref, w_ref, dinv_ref, hp_ref):
        hp_ref[...] = jnp.dot(x_ref[...], w_ref[...],
                              preferred_element_type=jnp.float32,
                              precision=lax.Precision.HIGHEST) * dinv_ref[...]

    return pl.pallas_call(
        body,
        out_shape=jax.ShapeDtypeStruct((n, h), jnp.float32),
    )(xin, w, dinv)


def kernel(x, edge_index, W0, b0, g0, be0, W1, b1, g1, be1, W2, b2, g2, be2):
    n, d = x.shape
    e = edge_index.shape[1]

    epad = -(-e // (_NW * _C * 8)) * (_NW * _C * 8)
    npad = -(-(n + 1) // (_NS * 8)) * (_NS * 8)
    rpt = npad // _NS

    src = edge_index[0]
    dst = edge_index[1]
    pad = epad - e
    srcp = jnp.concatenate([src, jnp.zeros((pad,), src.dtype)])
    dstp = jnp.concatenate([dst, jnp.full((pad,), n, dst.dtype)])
    src2 = srcp.reshape(epad // _C, _C)
    dst2 = dstp.reshape(epad // _C, _C)
    zrow = jnp.zeros((rpt, d), jnp.float32)
    ones128 = jnp.ones((_C, d), jnp.float32)

    degp = _sc_degree(dst2, ones128, zrow)
    dinv, hp0 = _tc_init(x, W0, degp)
    p0 = _sc_edge(hp0, src2, dst2, zrow)
    x1 = _tc_bn(p0, hp0, dinv, x, b0, g0, be0)
    hp1 = _tc_mm(x1, W1, dinv)
    p1 = _sc_edge(hp1, src2, dst2, zrow)
    x2 = _tc_bn(p1, hp1, dinv, x1, b1, g1, be1)
    hp2 = _tc_mm(x2, W2, dinv)
    p2 = _sc_edge(hp2, src2, dst2, zrow)
    return _tc_bn(p2, hp2, dinv, x2, b2, g2, be2)


# speedup vs baseline: 1.3710x; 1.3710x over previous
import functools

import jax
import jax.numpy as jnp
from jax import lax
from jax.experimental import pallas as pl
from jax.experimental.pallas import tpu as pltpu
from jax.experimental.pallas import tpu_sc as plsc

_NC = 2
_NS = 16
_NW = _NC * _NS
_C = 128
_EPS = 1e-5


def _mesh():
    return plsc.VectorSubcoreMesh(core_axis_name="c", subcore_axis_name="s")


def _sc_degree(dst2, ones_blk, zrow):
    cpw = dst2.shape[0] // _NW
    rpt = zrow.shape[0]
    npad = rpt * _NS
    dh = zrow.shape[1]

    @functools.partial(
        pl.kernel,
        out_type=jax.ShapeDtypeStruct((_NC, npad, dh), jnp.float32),
        mesh=_mesh(),
        scratch_types=[
            pltpu.VMEM((cpw, _C), jnp.int32),
            pltpu.VMEM((_C, dh), jnp.float32),
            pltpu.VMEM_SHARED((npad, dh), jnp.float32),
        ],
    )
    def degk(dst_hbm, ones_hbm, z_hbm, out_hbm, dstv, onev, acc):
        cid = lax.axis_index("c")
        sid = lax.axis_index("s")
        wid = sid * _NC + cid
        pltpu.sync_copy(z_hbm, acc.at[pl.ds(sid * rpt, rpt)])
        pltpu.sync_copy(ones_hbm, onev)
        pltpu.sync_copy(dst_hbm.at[pl.ds(wid * cpw, cpw)], dstv)
        plsc.subcore_barrier()

        @pl.loop(0, cpw)
        def _(ci):
            pltpu.sync_copy(onev, acc.at[dstv.at[ci]], add=True)

        plsc.subcore_barrier()
        pltpu.sync_copy(acc.at[pl.ds(sid * rpt, rpt)],
                        out_hbm.at[cid, pl.ds(sid * rpt, rpt)])

    return degk(dst2, ones_blk, zrow)


def _sc_edge(hp, src2, dst2, zrow):
    dh = hp.shape[1]
    cpw = src2.shape[0] // _NW
    rpt = zrow.shape[0]
    npad = rpt * _NS
    cpf = 3 * cpw // 2
    cps = cpw // 2

    @functools.partial(
        pl.kernel,
        out_type=jax.ShapeDtypeStruct((_NC, npad, dh), jnp.float32),
        mesh=_mesh(),
        scratch_types=[
            pltpu.VMEM((cpf, _C), jnp.int32),
            pltpu.VMEM((_C,), jnp.int32),
            pltpu.VMEM((_C,), jnp.int32),
            pltpu.VMEM((_C, dh), jnp.float32),
            pltpu.VMEM((_C, dh), jnp.float32),
            pltpu.VMEM_SHARED((npad, dh), jnp.float32),
            pltpu.SemaphoreType.DMA,
            pltpu.SemaphoreType.DMA,
            pltpu.SemaphoreType.DMA,
            pltpu.SemaphoreType.DMA,
        ],
    )
    def ek(hp_hbm, src_hbm, dst_hbm, z_hbm, out_hbm,
           dstv, s0, s1, buf0, buf1, acc, g0, g1, p0, p1):
        cid = lax.axis_index("c")
        sid = lax.axis_index("s")
        mycpw = jnp.where(cid == 0, cpf, cps)
        base = jnp.where(cid == 0, sid * cpf, _NS * cpf + sid * cps)
        pltpu.sync_copy(z_hbm, acc.at[pl.ds(sid * rpt, rpt)])
        pltpu.sync_copy(dst_hbm.at[pl.ds(base, cps)], dstv.at[pl.ds(0, cps)])

        @pl.when(cid == 0)
        def _():
            pltpu.sync_copy(dst_hbm.at[pl.ds(base + cps, cpf - cps)],
                            dstv.at[pl.ds(cps, cpf - cps)])

        plsc.subcore_barrier()

        hc = _C // 2

        @pl.loop(0, mycpw, step=2)
        def _(ci):
            pltpu.sync_copy(src_hbm.at[base + ci], s0)
            da0 = pltpu.async_copy(hp_hbm.at[s0.at[pl.ds(0, hc)]],
                                   buf0.at[pl.ds(0, hc)], g0)
            da1 = pltpu.async_copy(hp_hbm.at[s0.at[pl.ds(hc, hc)]],
                                   buf0.at[pl.ds(hc, hc)], p0)
            pltpu.sync_copy(src_hbm.at[base + ci + 1], s1)
            db0 = pltpu.async_copy(hp_hbm.at[s1.at[pl.ds(0, hc)]],
                                   buf1.at[pl.ds(0, hc)], g1)
            db1 = pltpu.async_copy(hp_hbm.at[s1.at[pl.ds(hc, hc)]],
                                   buf1.at[pl.ds(hc, hc)], p1)
            da0.wait()
            da1.wait()
            pltpu.sync_copy(buf0, acc.at[dstv.at[ci]], add=True)
            db0.wait()
            db1.wait()
            pltpu.sync_copy(buf1, acc.at[dstv.at[ci + 1]], add=True)

        plsc.subcore_barrier()
        pltpu.sync_copy(acc.at[pl.ds(sid * rpt, rpt)],
                        out_hbm.at[cid, pl.ds(sid * rpt, rpt)])

    return ek(hp, src2, dst2, zrow)


def _tc_init(x, w0, degp):
    n, d = x.shape
    h = w0.shape[1]

    def body(x_ref, w_ref, dp_ref, dinv_ref, hp_ref):
        deg = 1.0 + dp_ref[0][:n, 0:1] + dp_ref[1][:n, 0:1]
        dinv = 1.0 / jnp.sqrt(deg)
        dinv_ref[...] = dinv
        hp_ref[...] = jnp.dot(x_ref[...], w_ref[...],
                              preferred_element_type=jnp.float32,
                              precision=lax.Precision.HIGHEST) * dinv

    return pl.pallas_call(
        body,
        out_shape=(jax.ShapeDtypeStruct((n, 1), jnp.float32),
                   jax.ShapeDtypeStruct((n, h), jnp.float32)),
    )(x, w0, degp)


def _bn_relu_res(p_ref, hp, dinv, xres, b, g, be):
    n = hp.shape[0]
    scat = p_ref[0][:n, :] + p_ref[1][:n, :]
    t = dinv * (scat + hp) + b
    m = jnp.mean(t, axis=0, keepdims=True)
    v = jnp.mean((t - m) ** 2, axis=0, keepdims=True)
    hh = g * (t - m) / jnp.sqrt(v + _EPS) + be
    return xres + jnp.maximum(hh, 0.0)


def _tc_bn(p, hp, dinv, xres, b, g, be):
    n, h = hp.shape

    def body(p_ref, hp_ref, dinv_ref, x_ref, b_ref, g_ref, be_ref, xn_ref):
        xn_ref[...] = _bn_relu_res(p_ref, hp_ref[...], dinv_ref[...],
                                   x_ref[...], b_ref[...], g_ref[...],
                                   be_ref[...])

    return pl.pallas_call(
        body,
        out_shape=jax.ShapeDtypeStruct((n, h), jnp.float32),
    )(p, hp, dinv, xres, b, g, be)


def _tc_mm(xin, w, dinv):
    n, h = xin.shape[0], w.shape[1]

    def body(x_ref, w_ref, dinv_ref, hp_ref):
        hp_ref[...] = jnp.dot(x_ref[...], w_ref[...],
                              preferred_element_type=jnp.float32,
                              precision=lax.Precision.HIGHEST) * dinv_ref[...]

    return pl.pallas_call(
        body,
        out_shape=jax.ShapeDtypeStruct((n, h), jnp.float32),
    )(xin, w, dinv)


def kernel(x, edge_index, W0, b0, g0, be0, W1, b1, g1, be1, W2, b2, g2, be2):
    n, d = x.shape
    e = edge_index.shape[1]

    epad = -(-e // (_NW * _C * 8)) * (_NW * _C * 8)
    npad = -(-(n + 1) // (_NS * 8)) * (_NS * 8)
    rpt = npad // _NS

    src = edge_index[0]
    dst = edge_index[1]
    pad = epad - e
    srcp = jnp.concatenate([src, jnp.zeros((pad,), src.dtype)])
    dstp = jnp.concatenate([dst, jnp.full((pad,), n, dst.dtype)])
    src2 = srcp.reshape(epad // _C, _C)
    dst2 = dstp.reshape(epad // _C, _C)
    zrow = jnp.zeros((rpt, d), jnp.float32)
    ones128 = jnp.ones((_C, d), jnp.float32)

    degp = _sc_degree(dst2, ones128, zrow)
    dinv, hp0 = _tc_init(x, W0, degp)
    p0 = _sc_edge(hp0, src2, dst2, zrow)
    x1 = _tc_bn(p0, hp0, dinv, x, b0, g0, be0)
    hp1 = _tc_mm(x1, W1, dinv)
    p1 = _sc_edge(hp1, src2, dst2, zrow)
    x2 = _tc_bn(p1, hp1, dinv, x1, b1, g1, be1)
    hp2 = _tc_mm(x2, W2, dinv)
    p2 = _sc_edge(hp2, src2, dst2, zrow)
    return _tc_bn(p2, hp2, dinv, x2, b2, g2, be2)


# speedup vs baseline: 1.4079x; 1.0270x over previous
import functools

import jax
import jax.numpy as jnp
from jax import lax
from jax.experimental import pallas as pl
from jax.experimental.pallas import tpu as pltpu
from jax.experimental.pallas import tpu_sc as plsc

_NC = 2
_NS = 16
_NW = _NC * _NS
_C = 128
_EPS = 1e-5


def _mesh():
    return plsc.VectorSubcoreMesh(core_axis_name="c", subcore_axis_name="s")


def _sc_degree(dst2, ones_blk, zrow):
    cpw = dst2.shape[0] // _NW
    rpt = zrow.shape[0]
    npad = rpt * _NS
    dh = zrow.shape[1]

    @functools.partial(
        pl.kernel,
        out_type=jax.ShapeDtypeStruct((_NC, npad, dh), jnp.float32),
        mesh=_mesh(),
        scratch_types=[
            pltpu.VMEM((cpw, _C), jnp.int32),
            pltpu.VMEM((_C, dh), jnp.float32),
            pltpu.VMEM_SHARED((npad, dh), jnp.float32),
        ],
    )
    def degk(dst_hbm, ones_hbm, z_hbm, out_hbm, dstv, onev, acc):
        cid = lax.axis_index("c")
        sid = lax.axis_index("s")
        wid = sid * _NC + cid
        pltpu.sync_copy(z_hbm, acc.at[pl.ds(sid * rpt, rpt)])
        pltpu.sync_copy(ones_hbm, onev)
        pltpu.sync_copy(dst_hbm.at[pl.ds(wid * cpw, cpw)], dstv)
        plsc.subcore_barrier()

        @pl.loop(0, cpw)
        def _(ci):
            pltpu.sync_copy(onev, acc.at[dstv.at[ci]], add=True)

        plsc.subcore_barrier()
        pltpu.sync_copy(acc.at[pl.ds(sid * rpt, rpt)],
                        out_hbm.at[cid, pl.ds(sid * rpt, rpt)])

    return degk(dst2, ones_blk, zrow)


def _sc_edge(hp, src2, dst2, zrow):
    dh = hp.shape[1]
    cpw = src2.shape[0] // _NW
    rpt = zrow.shape[0]
    npad = rpt * _NS
    cpf = 8 * cpw // 5
    cps = 2 * cpw // 5

    @functools.partial(
        pl.kernel,
        out_type=jax.ShapeDtypeStruct((_NC, npad, dh), jnp.float32),
        mesh=_mesh(),
        scratch_types=[
            pltpu.VMEM((cpf, _C), jnp.int32),
            pltpu.VMEM((_C,), jnp.int32),
            pltpu.VMEM((_C,), jnp.int32),
            pltpu.VMEM((_C, dh), jnp.float32),
            pltpu.VMEM((_C, dh), jnp.float32),
            pltpu.VMEM_SHARED((npad, dh), jnp.float32),
            pltpu.SemaphoreType.DMA,
            pltpu.SemaphoreType.DMA,
            pltpu.SemaphoreType.DMA,
            pltpu.SemaphoreType.DMA,
        ],
    )
    def ek(hp_hbm, src_hbm, dst_hbm, z_hbm, out_hbm,
           dstv, s0, s1, buf0, buf1, acc, g0, g1, p0, p1):
        cid = lax.axis_index("c")
        sid = lax.axis_index("s")
        mycpw = jnp.where(cid == 0, cpf, cps)
        base = jnp.where(cid == 0, sid * cpf, _NS * cpf + sid * cps)
        pltpu.sync_copy(z_hbm, acc.at[pl.ds(sid * rpt, rpt)])
        pltpu.sync_copy(dst_hbm.at[pl.ds(base, cps)], dstv.at[pl.ds(0, cps)])

        @pl.when(cid == 0)
        def _():
            pltpu.sync_copy(dst_hbm.at[pl.ds(base + cps, cpf - cps)],
                            dstv.at[pl.ds(cps, cpf - cps)])

        plsc.subcore_barrier()

        hc = _C // 2

        @pl.loop(0, mycpw, step=2)
        def _(ci):
            pltpu.sync_copy(src_hbm.at[base + ci], s0)
            da0 = pltpu.async_copy(hp_hbm.at[s0.at[pl.ds(0, hc)]],
                                   buf0.at[pl.ds(0, hc)], g0)
            da1 = pltpu.async_copy(hp_hbm.at[s0.at[pl.ds(hc, hc)]],
                                   buf0.at[pl.ds(hc, hc)], p0)
            pltpu.sync_copy(src_hbm.at[base + ci + 1], s1)
            db0 = pltpu.async_copy(hp_hbm.at[s1.at[pl.ds(0, hc)]],
                                   buf1.at[pl.ds(0, hc)], g1)
            db1 = pltpu.async_copy(hp_hbm.at[s1.at[pl.ds(hc, hc)]],
                                   buf1.at[pl.ds(hc, hc)], p1)
            da0.wait()
            da1.wait()
            pltpu.sync_copy(buf0, acc.at[dstv.at[ci]], add=True)
            db0.wait()
            db1.wait()
            pltpu.sync_copy(buf1, acc.at[dstv.at[ci + 1]], add=True)

        plsc.subcore_barrier()
        pltpu.sync_copy(acc.at[pl.ds(sid * rpt, rpt)],
                        out_hbm.at[cid, pl.ds(sid * rpt, rpt)])

    return ek(hp, src2, dst2, zrow)


def _tc_init(x, w0, degp):
    n, d = x.shape
    h = w0.shape[1]

    def body(x_ref, w_ref, dp_ref, dinv_ref, hp_ref):
        deg = 1.0 + dp_ref[0][:n, 0:1] + dp_ref[1][:n, 0:1]
        dinv = 1.0 / jnp.sqrt(deg)
        dinv_ref[...] = dinv
        hp_ref[...] = jnp.dot(x_ref[...], w_ref[...],
                              preferred_element_type=jnp.float32,
                              precision=lax.Precision.HIGHEST) * dinv

    return pl.pallas_call(
        body,
        out_shape=(jax.ShapeDtypeStruct((n, 1), jnp.float32),
                   jax.ShapeDtypeStruct((n, h), jnp.float32)),
    )(x, w0, degp)


def _bn_relu_res(p_ref, hp, dinv, xres, b, g, be):
    n = hp.shape[0]
    scat = p_ref[0][:n, :] + p_ref[1][:n, :]
    t = dinv * (scat + hp) + b
    m = jnp.mean(t, axis=0, keepdims=True)
    v = jnp.mean((t - m) ** 2, axis=0, keepdims=True)
    hh = g * (t - m) / jnp.sqrt(v + _EPS) + be
    return xres + jnp.maximum(hh, 0.0)


def _tc_bn(p, hp, dinv, xres, b, g, be):
    n, h = hp.shape

    def body(p_ref, hp_ref, dinv_ref, x_ref, b_ref, g_ref, be_ref, xn_ref):
        xn_ref[...] = _bn_relu_res(p_ref, hp_ref[...], dinv_ref[...],
                                   x_ref[...], b_ref[...], g_ref[...],
                                   be_ref[...])

    return pl.pallas_call(
        body,
        out_shape=jax.ShapeDtypeStruct((n, h), jnp.float32),
    )(p, hp, dinv, xres, b, g, be)


def _tc_mm(xin, w, dinv):
    n, h = xin.shape[0], w.shape[1]

    def body(x_ref, w_ref, dinv_ref, hp_ref):
        hp_ref[...] = jnp.dot(x_ref[...], w_ref[...],
                              preferred_element_type=jnp.float32,
                              precision=lax.Precision.HIGHEST) * dinv_ref[...]

    return pl.pallas_call(
        body,
        out_shape=jax.ShapeDtypeStruct((n, h), jnp.float32),
    )(xin, w, dinv)


def kernel(x, edge_index, W0, b0, g0, be0, W1, b1, g1, be1, W2, b2, g2, be2):
    n, d = x.shape
    e = edge_index.shape[1]

    epad = -(-e // (_NW * _C * 8)) * (_NW * _C * 8)
    npad = -(-(n + 1) // (_NS * 8)) * (_NS * 8)
    rpt = npad // _NS

    src = edge_index[0]
    dst = edge_index[1]
    pad = epad - e
    srcp = jnp.concatenate([src, jnp.zeros((pad,), src.dtype)])
    dstp = jnp.concatenate([dst, jnp.full((pad,), n, dst.dtype)])
    src2 = srcp.reshape(epad // _C, _C)
    dst2 = dstp.reshape(epad // _C, _C)
    zrow = jnp.zeros((rpt, d), jnp.float32)
    ones128 = jnp.ones((_C, d), jnp.float32)

    degp = _sc_degree(dst2, ones128, zrow)
    dinv, hp0 = _tc_init(x, W0, degp)
    p0 = _sc_edge(hp0, src2, dst2, zrow)
    x1 = _tc_bn(p0, hp0, dinv, x, b0, g0, be0)
    hp1 = _tc_mm(x1, W1, dinv)
    p1 = _sc_edge(hp1, src2, dst2, zrow)
    x2 = _tc_bn(p1, hp1, dinv, x1, b1, g1, be1)
    hp2 = _tc_mm(x2, W2, dinv)
    p2 = _sc_edge(hp2, src2, dst2, zrow)
    return _tc_bn(p2, hp2, dinv, x2, b2, g2, be2)


# speedup vs baseline: 1.4924x; 1.0600x over previous
import functools

import jax
import jax.numpy as jnp
from jax import lax
from jax.experimental import pallas as pl
from jax.experimental.pallas import tpu as pltpu
from jax.experimental.pallas import tpu_sc as plsc

_NC = 2
_NS = 16
_NW = _NC * _NS
_C = 128
_EPS = 1e-5


def _mesh():
    return plsc.VectorSubcoreMesh(core_axis_name="c", subcore_axis_name="s")


def _sc_degree(dst2, ones_blk, zrow):
    cpw = dst2.shape[0] // _NW
    rpt = zrow.shape[0]
    npad = rpt * _NS
    dh = zrow.shape[1]

    @functools.partial(
        pl.kernel,
        out_type=jax.ShapeDtypeStruct((_NC, npad, dh), jnp.float32),
        mesh=_mesh(),
        scratch_types=[
            pltpu.VMEM((cpw, _C), jnp.int32),
            pltpu.VMEM((_C, dh), jnp.float32),
            pltpu.VMEM_SHARED((npad, dh), jnp.float32),
        ],
    )
    def degk(dst_hbm, ones_hbm, z_hbm, out_hbm, dstv, onev, acc):
        cid = lax.axis_index("c")
        sid = lax.axis_index("s")
        wid = sid * _NC + cid
        pltpu.sync_copy(z_hbm, acc.at[pl.ds(sid * rpt, rpt)])
        pltpu.sync_copy(ones_hbm, onev)
        pltpu.sync_copy(dst_hbm.at[pl.ds(wid * cpw, cpw)], dstv)
        plsc.subcore_barrier()

        @pl.loop(0, cpw)
        def _(ci):
            pltpu.sync_copy(onev, acc.at[dstv.at[ci]], add=True)

        plsc.subcore_barrier()
        pltpu.sync_copy(acc.at[pl.ds(sid * rpt, rpt)],
                        out_hbm.at[cid, pl.ds(sid * rpt, rpt)])

    return degk(dst2, ones_blk, zrow)


def _sc_edge(hp, src2, dst2, zrow):
    dh = hp.shape[2]
    cpw = src2.shape[0] // _NW
    rpt = zrow.shape[0]
    npad = rpt * _NS
    cpf = 8 * cpw // 5
    cps = 2 * cpw // 5

    @functools.partial(
        pl.kernel,
        out_type=jax.ShapeDtypeStruct((_NC, npad, dh), jnp.float32),
        mesh=_mesh(),
        scratch_types=[
            pltpu.VMEM((cpf, _C), jnp.int32),
            pltpu.VMEM((_C,), jnp.int32),
            pltpu.VMEM((_C,), jnp.int32),
            pltpu.VMEM((_C, dh), jnp.float32),
            pltpu.VMEM((_C, dh), jnp.float32),
            pltpu.VMEM_SHARED((npad, dh), jnp.float32),
            pltpu.SemaphoreType.DMA,
            pltpu.SemaphoreType.DMA,
            pltpu.SemaphoreType.DMA,
            pltpu.SemaphoreType.DMA,
        ],
    )
    def ek(hp_hbm, src_hbm, dst_hbm, z_hbm, out_hbm,
           dstv, s0, s1, buf0, buf1, acc, g0, g1, p0, p1):
        cid = lax.axis_index("c")
        sid = lax.axis_index("s")
        table = hp_hbm.at[cid]
        mycpw = jnp.where(cid == 0, cpf, cps)
        base = jnp.where(cid == 0, sid * cpf, _NS * cpf + sid * cps)
        pltpu.sync_copy(z_hbm, acc.at[pl.ds(sid * rpt, rpt)])
        pltpu.sync_copy(dst_hbm.at[pl.ds(base, cps)], dstv.at[pl.ds(0, cps)])

        @pl.when(cid == 0)
        def _():
            pltpu.sync_copy(dst_hbm.at[pl.ds(base + cps, cpf - cps)],
                            dstv.at[pl.ds(cps, cpf - cps)])

        plsc.subcore_barrier()

        hc = _C // 2

        @pl.loop(0, mycpw, step=2)
        def _(ci):
            pltpu.sync_copy(src_hbm.at[base + ci], s0)
            da0 = pltpu.async_copy(table.at[s0.at[pl.ds(0, hc)]],
                                   buf0.at[pl.ds(0, hc)], g0)
            da1 = pltpu.async_copy(table.at[s0.at[pl.ds(hc, hc)]],
                                   buf0.at[pl.ds(hc, hc)], p0)
            pltpu.sync_copy(src_hbm.at[base + ci + 1], s1)
            db0 = pltpu.async_copy(table.at[s1.at[pl.ds(0, hc)]],
                                   buf1.at[pl.ds(0, hc)], g1)
            db1 = pltpu.async_copy(table.at[s1.at[pl.ds(hc, hc)]],
                                   buf1.at[pl.ds(hc, hc)], p1)
            da0.wait()
            da1.wait()
            pltpu.sync_copy(buf0, acc.at[dstv.at[ci]], add=True)
            db0.wait()
            db1.wait()
            pltpu.sync_copy(buf1, acc.at[dstv.at[ci + 1]], add=True)

        plsc.subcore_barrier()
        pltpu.sync_copy(acc.at[pl.ds(sid * rpt, rpt)],
                        out_hbm.at[cid, pl.ds(sid * rpt, rpt)])

    return ek(hp, src2, dst2, zrow)


def _tc_init(x, w0, degp):
    n, d = x.shape
    h = w0.shape[1]

    def body(x_ref, w_ref, dp_ref, dinv_ref, hp_ref):
        deg = 1.0 + dp_ref[0][:n, 0:1] + dp_ref[1][:n, 0:1]
        dinv = 1.0 / jnp.sqrt(deg)
        dinv_ref[...] = dinv
        hp = jnp.dot(x_ref[...], w_ref[...],
                     preferred_element_type=jnp.float32,
                     precision=lax.Precision.HIGHEST) * dinv
        hp_ref[0] = hp
        hp_ref[1] = hp

    return pl.pallas_call(
        body,
        out_shape=(jax.ShapeDtypeStruct((n, 1), jnp.float32),
                   jax.ShapeDtypeStruct((2, n, h), jnp.float32)),
    )(x, w0, degp)


def _bn_relu_res(p_ref, hp, dinv, xres, b, g, be):
    n = hp.shape[0]
    scat = p_ref[0][:n, :] + p_ref[1][:n, :]
    t = dinv * (scat + hp) + b
    m = jnp.mean(t, axis=0, keepdims=True)
    v = jnp.mean((t - m) ** 2, axis=0, keepdims=True)
    hh = g * (t - m) / jnp.sqrt(v + _EPS) + be
    return xres + jnp.maximum(hh, 0.0)


def _tc_bn(p, hp, dinv, xres, b, g, be):
    n, h = hp.shape

    def body(p_ref, hp_ref, dinv_ref, x_ref, b_ref, g_ref, be_ref, xn_ref):
        xn_ref[...] = _bn_relu_res(p_ref, hp_ref[...], dinv_ref[...],
                                   x_ref[...], b_ref[...], g_ref[...],
                                   be_ref[...])

    return pl.pallas_call(
        body,
        out_shape=jax.ShapeDtypeStruct((n, h), jnp.float32),
    )(p, hp, dinv, xres, b, g, be)


def _tc_mm(xin, w, dinv):
    n, h = xin.shape[0], w.shape[1]

    def body(x_ref, w_ref, dinv_ref, hp_ref):
        hp = jnp.dot(x_ref[...], w_ref[...],
                     preferred_element_type=jnp.float32,
                     precision=lax.Precision.HIGHEST) * dinv_ref[...]
        hp_ref[0] = hp
        hp_ref[1] = hp

    return pl.pallas_call(
        body,
        out_shape=jax.ShapeDtypeStruct((2, n, h), jnp.float32),
    )(xin, w, dinv)


def kernel(x, edge_index, W0, b0, g0, be0, W1, b1, g1, be1, W2, b2, g2, be2):
    n, d = x.shape
    e = edge_index.shape[1]

    epad = -(-e // (_NW * _C * 8)) * (_NW * _C * 8)
    npad = -(-(n + 1) // (_NS * 8)) * (_NS * 8)
    rpt = npad // _NS

    src = edge_index[0]
    dst = edge_index[1]
    pad = epad - e
    srcp = jnp.concatenate([src, jnp.zeros((pad,), src.dtype)])
    dstp = jnp.concatenate([dst, jnp.full((pad,), n, dst.dtype)])
    src2 = srcp.reshape(epad // _C, _C)
    dst2 = dstp.reshape(epad // _C, _C)
    zrow = jnp.zeros((rpt, d), jnp.float32)
    ones128 = jnp.ones((_C, d), jnp.float32)

    degp = _sc_degree(dst2, ones128, zrow)
    dinv, hp0 = _tc_init(x, W0, degp)
    p0 = _sc_edge(hp0, src2, dst2, zrow)
    x1 = _tc_bn(p0, hp0[0], dinv, x, b0, g0, be0)
    hp1 = _tc_mm(x1, W1, dinv)
    p1 = _sc_edge(hp1, src2, dst2, zrow)
    x2 = _tc_bn(p1, hp1[0], dinv, x1, b1, g1, be1)
    hp2 = _tc_mm(x2, W2, dinv)
    p2 = _sc_edge(hp2, src2, dst2, zrow)
    return _tc_bn(p2, hp2[0], dinv, x2, b2, g2, be2)
